# baseline (speedup 1.0000x reference)
import functools

import jax
import jax.numpy as jnp
from jax import lax
from jax.experimental import pallas as pl
from jax.experimental.pallas import tpu as pltpu
from jax.experimental.pallas import tpu_sc as plsc

N = 10000
E = 320000
D = 128
H = 128
ED = 4
EDP = 8
M = 48000
NC = 2
NS = 16
NW = NC * NS
EROWS = E // 128
MROWS = M // 128
NP_ = 10112
RPT = NP_ // NS
ZR = 128
RCH = (128, 128, 128, 128, 120)
RCH_TAIL = (128, 128, 128, 128, 8)


def _sc_aggregate(x, src1d, dst1d, ea_pad, corrmat, mi1d, dm1d, z8):
    mesh = plsc.VectorSubcoreMesh(core_axis_name="c", subcore_axis_name="s")

    @functools.partial(
        pl.kernel,
        mesh=mesh,
        compiler_params=pltpu.CompilerParams(use_tc_tiling_on_sc=False),
        out_type=[
            jax.ShapeDtypeStruct((NC, N, D), jnp.float32),
            jax.ShapeDtypeStruct((NC, N, EDP), jnp.float32),
            jax.ShapeDtypeStruct((M, EDP), jnp.float32),
        ],
        scratch_types=[
            pltpu.VMEM((128,), jnp.int32),
            pltpu.VMEM((128,), jnp.int32),
            pltpu.VMEM((128, D), jnp.float32),
            pltpu.VMEM((128, EDP), jnp.float32),
            pltpu.VMEM((128,), jnp.int32),
            pltpu.VMEM((128,), jnp.int32),
            pltpu.VMEM((128, D), jnp.float32),
            pltpu.VMEM((128, EDP), jnp.float32),
            pltpu.VMEM_SHARED((NP_, D), jnp.float32),
            pltpu.VMEM_SHARED((NP_, EDP), jnp.float32),
            pltpu.SemaphoreType.DMA,
            pltpu.SemaphoreType.DMA,
            pltpu.SemaphoreType.DMA,
            pltpu.SemaphoreType.DMA,
            pltpu.SemaphoreType.DMA,
            pltpu.SemaphoreType.DMA,
        ],
    )
    def k(x_hbm, src_hbm, dst_hbm, ea_hbm, corr_hbm, mi_hbm, dm_hbm, z8_hbm,
          out_px, out_pea, out_orig,
          srcv0, dstv0, xrows0, earows0, srcv1, dstv1, xrows1, earows1,
          acc_x, acc_ea, semi0, semi1, semg0, semg1, sems0, sems1):
        c = lax.axis_index("c")
        s = lax.axis_index("s")
        wid = s * NC + c
        r0 = s * RPT

        bufs = (
            (srcv0, dstv0, xrows0, earows0, semi0, semg0, sems0),
            (srcv1, dstv1, xrows1, earows1, semi1, semg1, sems1),
        )

        zvec = jnp.zeros((16,), jnp.float32)

        def zrow(r, carry):
            for cc in range(D // 16):
                xrows0[r, pl.ds(cc * 16, 16)] = zvec
            return carry

        lax.fori_loop(0, ZR, zrow, 0)
        pltpu.sync_copy(z8_hbm, earows0)

        off = 0
        for ch in RCH:
            pltpu.sync_copy(xrows0.at[pl.ds(0, ch)],
                            acc_x.at[pl.ds(r0 + off, ch)])
            pltpu.sync_copy(earows0.at[pl.ds(0, ch)],
                            acc_ea.at[pl.ds(r0 + off, ch)])
            off += ch
        plsc.subcore_barrier()

        def issue_idx(t, b):
            rid = wid + NW * t

            @pl.when(rid < EROWS)
            def _():
                sv, dv, xr, ea, s_i, s_g, s_s = bufs[b]
                e0 = rid * 128
                pltpu.async_copy(src_hbm.at[pl.ds(e0, 128)], sv, s_i)
                pltpu.async_copy(dst_hbm.at[pl.ds(e0, 128)], dv, s_i)
                pltpu.async_copy(ea_hbm.at[pl.ds(e0, 128)], ea, s_i)

        issue_idx(0, 0)

        def pipe_pair(h, carry):
            for b in (0, 1):
                t = 2 * h + b
                nb = 1 - b
                sv, dv, xr, ea, s_i, s_g, s_s = bufs[b]
                svn, dvn, xrn, ean, s_in, s_gn, s_sn = bufs[nb]
                rid = wid + NW * t
                e0 = rid * 128

                @pl.when(rid < EROWS)
                def _():
                    pltpu.make_async_copy(
                        src_hbm.at[pl.ds(e0, 128)], sv, s_i).wait()
                    pltpu.make_async_copy(
                        dst_hbm.at[pl.ds(e0, 128)], dv, s_i).wait()
                    pltpu.make_async_copy(
                        ea_hbm.at[pl.ds(e0, 128)], ea, s_i).wait()
                    pltpu.async_copy(x_hbm.at[sv], xr, s_g)

                prev = wid + NW * (t - 1)

                @pl.when((t >= 1) & (prev < EROWS))
                def _():
                    pltpu.make_async_copy(xrn, acc_x.at[dvn], s_sn).wait()
                    pltpu.make_async_copy(ean, acc_ea.at[dvn], s_sn).wait()

                issue_idx(t + 1, nb)

                @pl.when(rid < EROWS)
                def _():
                    pltpu.make_async_copy(x_hbm.at[sv], xr, s_g).wait()
                    pltpu.async_copy(xr, acc_x.at[dv], s_s, add=True)
                    pltpu.async_copy(ea, acc_ea.at[dv], s_s, add=True)
            return carry

        lax.fori_loop(0, (EROWS // NW + 2) // 2, pipe_pair, 0)

        def mask_step(t, carry):
            rid = wid + NW * t

            @pl.when(rid < MROWS)
            def _():
                pltpu.sync_copy(mi_hbm.at[pl.ds(rid * 128, 128)], srcv0)
                pltpu.sync_copy(dm_hbm.at[pl.ds(rid * 128, 128)], dstv0)
                cp1 = pltpu.async_copy(ea_hbm.at[srcv0], earows0, semg0)
                cp2 = pltpu.async_copy(corr_hbm.at[srcv0], earows1, semg1)
                cp1.wait()
                cp2.wait()
                pltpu.sync_copy(earows0, out_orig.at[pl.ds(rid * 128, 128)])
                pltpu.sync_copy(earows1, acc_ea.at[dstv0], add=True)

            return carry

        lax.fori_loop(0, (MROWS + NW - 1) // NW, mask_step, 0)

        plsc.subcore_barrier()

        @pl.when(s < NS - 1)
        def _():
            off = 0
            for ch in RCH:
                pltpu.sync_copy(acc_x.at[pl.ds(r0 + off, ch)],
                                xrows0.at[pl.ds(0, ch)])
                pltpu.sync_copy(xrows0.at[pl.ds(0, ch)],
                                out_px.at[c, pl.ds(r0 + off, ch)])
                pltpu.sync_copy(acc_ea.at[pl.ds(r0 + off, ch)],
                                earows0.at[pl.ds(0, ch)])
                pltpu.sync_copy(earows0.at[pl.ds(0, ch)],
                                out_pea.at[c, pl.ds(r0 + off, ch)])
                off += ch

        @pl.when(s == NS - 1)
        def _():
            off = 0
            for ch in RCH_TAIL:
                pltpu.sync_copy(acc_x.at[pl.ds(r0 + off, ch)],
                                xrows0.at[pl.ds(0, ch)])
                pltpu.sync_copy(xrows0.at[pl.ds(0, ch)],
                                out_px.at[c, pl.ds(r0 + off, ch)])
                pltpu.sync_copy(acc_ea.at[pl.ds(r0 + off, ch)],
                                earows0.at[pl.ds(0, ch)])
                pltpu.sync_copy(earows0.at[pl.ds(0, ch)],
                                out_pea.at[c, pl.ds(r0 + off, ch)])
                off += ch

    return k(x, src1d, dst1d, ea_pad, corrmat, mi1d, dm1d, z8)


def _sc_gather_pairs(p1, p2, sm1d, dm1d):
    mesh = plsc.VectorSubcoreMesh(core_axis_name="c", subcore_axis_name="s")

    @functools.partial(
        pl.kernel,
        mesh=mesh,
        out_type=[
            jax.ShapeDtypeStruct((M, H), jnp.float32),
            jax.ShapeDtypeStruct((M, H), jnp.float32),
        ],
        scratch_types=[
            pltpu.VMEM((128,), jnp.int32),
            pltpu.VMEM((128,), jnp.int32),
            pltpu.VMEM((128, H), jnp.float32),
            pltpu.VMEM((128, H), jnp.float32),
            pltpu.VMEM((128,), jnp.int32),
            pltpu.VMEM((128,), jnp.int32),
            pltpu.VMEM((128, H), jnp.float32),
            pltpu.VMEM((128, H), jnp.float32),
            pltpu.SemaphoreType.DMA,
            pltpu.SemaphoreType.DMA,
            pltpu.SemaphoreType.DMA,
            pltpu.SemaphoreType.DMA,
            pltpu.SemaphoreType.DMA,
            pltpu.SemaphoreType.DMA,
        ],
    )
    def k(p1_hbm, p2_hbm, sm_hbm, dm_hbm, g1_out, g2_out,
          smv0, dmv0, r10, r20, smv1, dmv1, r11, r21,
          semi0, semi1, semg0, semg1, semw0, semw1):
        c = lax.axis_index("c")
        s = lax.axis_index("s")
        wid = s * NC + c

        bufs = (
            (smv0, dmv0, r10, r20, semi0, semg0, semw0),
            (smv1, dmv1, r11, r21, semi1, semg1, semw1),
        )

        def issue_idx(t, b):
            rid = wid + NW * t

            @pl.when(rid < MROWS)
            def _():
                sm_, dm_, r1_, r2_, s_i, s_g, s_w = bufs[b]
                e0 = rid * 128
                pltpu.async_copy(sm_hbm.at[pl.ds(e0, 128)], sm_, s_i)
                pltpu.async_copy(dm_hbm.at[pl.ds(e0, 128)], dm_, s_i)

        issue_idx(0, 0)

        def pipe_pair(h, carry):
            for b in (0, 1):
                t = 2 * h + b
                nb = 1 - b
                sm_, dm_, r1_, r2_, s_i, s_g, s_w = bufs[b]
                smn, dmn, r1n, r2n, s_in, s_gn, s_wn = bufs[nb]
                rid = wid + NW * t
                e0 = rid * 128

                @pl.when(rid < MROWS)
                def _():
                    pltpu.make_async_copy(
                        sm_hbm.at[pl.ds(e0, 128)], sm_, s_i).wait()
                    pltpu.make_async_copy(
                        dm_hbm.at[pl.ds(e0, 128)], dm_, s_i).wait()
                    pltpu.async_copy(p1_hbm.at[sm_], r1_, s_g)
                    pltpu.async_copy(p2_hbm.at[dm_], r2_, s_g)

                prev = wid + NW * (t - 1)
                pe0 = prev * 128

                @pl.when((t >= 1) & (prev < MROWS))
                def _():
                    pltpu.make_async_copy(
                        r1n, g1_out.at[pl.ds(pe0, 128)], s_wn).wait()
                    pltpu.make_async_copy(
                        r2n, g2_out.at[pl.ds(pe0, 128)], s_wn).wait()

                issue_idx(t + 1, nb)

                @pl.when(rid < MROWS)
                def _():
                    pltpu.make_async_copy(p1_hbm.at[sm_], r1_, s_g).wait()
                    pltpu.make_async_copy(p2_hbm.at[dm_], r2_, s_g).wait()
                    pltpu.async_copy(r1_, g1_out.at[pl.ds(e0, 128)], s_w)
                    pltpu.async_copy(r2_, g2_out.at[pl.ds(e0, 128)], s_w)
            return carry

        lax.fori_loop(0, (MROWS // NW + 2) // 2, pipe_pair, 0)

    return k(p1, p2, sm1d, dm1d)


def _tc_encode(x, px0, px1, pea0, pea1, w_self, wmx, wme, w1a, w1b, b1):
    BN = 1000

    def body(x_r, px0_r, px1_r, pea0_r, pea1_r, ws_r, wmx_r, wme_r,
             w1a_r, w1b_r, b1_r, p1_o, p2_o):
        aggx = px0_r[...] + px1_r[...]
        aggea = pea0_r[...] + pea1_r[...]
        z = (jnp.dot(x_r[...], ws_r[...], preferred_element_type=jnp.float32)
             + jnp.dot(aggx, wmx_r[...], preferred_element_type=jnp.float32)
             + jnp.dot(aggea, wme_r[...], preferred_element_type=jnp.float32))
        emb = jnp.maximum(z, 0.0)
        p1_o[...] = (jnp.dot(emb, w1a_r[...], preferred_element_type=jnp.float32)
                     + b1_r[...])
        p2_o[...] = jnp.dot(emb, w1b_r[...], preferred_element_type=jnp.float32)

    row = lambda i: (i, 0)
    fix = lambda i: (0, 0)
    return pl.pallas_call(
        body,
        grid=(N // BN,),
        in_specs=[
            pl.BlockSpec((BN, D), row),
            pl.BlockSpec((BN, D), row),
            pl.BlockSpec((BN, D), row),
            pl.BlockSpec((BN, EDP), row),
            pl.BlockSpec((BN, EDP), row),
            pl.BlockSpec((D, H), fix),
            pl.BlockSpec((D, H), fix),
            pl.BlockSpec((EDP, H), fix),
            pl.BlockSpec((H, H), fix),
            pl.BlockSpec((H, H), fix),
            pl.BlockSpec((1, H), fix),
        ],
        out_specs=[
            pl.BlockSpec((BN, H), row),
            pl.BlockSpec((BN, H), row),
        ],
        out_shape=[
            jax.ShapeDtypeStruct((N, H), jnp.float32),
            jax.ShapeDtypeStruct((N, H), jnp.float32),
        ],
    )(x, px0, px1, pea0, pea1, w_self, wmx, wme, w1a, w1b, b1.reshape(1, H))


def _tc_loss(g1, g2, orig16, w2p, b2p):
    BM = 4800
    scale = 1.0 / (M * ED)

    def body(g1_r, g2_r, o_r, w2_r, b2_r, out_ref):
        i = pl.program_id(0)
        h = jnp.maximum(g1_r[...] + g2_r[...], 0.0)
        pred = (jnp.dot(h, w2_r[...], preferred_element_type=jnp.float32)
                + b2_r[...])
        part = jnp.sum(jnp.abs(pred - o_r[...])) * scale

        @pl.when(i == 0)
        def _():
            out_ref[0, 0] = part

        @pl.when(i > 0)
        def _():
            out_ref[0, 0] += part

    row = lambda i: (i, 0)
    fix = lambda i: (0, 0)
    out = pl.pallas_call(
        body,
        grid=(M // BM,),
        in_specs=[
            pl.BlockSpec((BM, H), row),
            pl.BlockSpec((BM, H), row),
            pl.BlockSpec((BM, EDP), row),
            pl.BlockSpec((H, EDP), fix),
            pl.BlockSpec((1, EDP), fix),
        ],
        out_specs=pl.BlockSpec((1, 1), fix, memory_space=pltpu.SMEM),
        out_shape=jax.ShapeDtypeStruct((1, 1), jnp.float32),
    )(g1, g2, orig16, w2p, b2p.reshape(1, EDP))
    return out[0, 0]


def kernel(x, edge_index, edge_attr, mask_indices, edge_mask_token,
           W_self, W_msg, W1, b1, W2, b2):
    src = edge_index[0]
    dst = edge_index[1]
    ea_pad = jnp.pad(edge_attr, ((0, 0), (0, EDP - ED)))
    tok_pad = jnp.pad(edge_mask_token, (0, EDP - ED))
    corrmat = tok_pad - ea_pad
    z8 = jnp.zeros((128, EDP), jnp.float32)
    sm = jnp.take(src, mask_indices, mode="clip")
    dm = jnp.take(dst, mask_indices, mode="clip")

    px, pea, orig16 = _sc_aggregate(x, src, dst, ea_pad, corrmat,
                                    mask_indices, dm, z8)

    wmx = W_msg[:D]
    wme = jnp.pad(W_msg[D:], ((0, EDP - ED), (0, 0)))
    p1, p2 = _tc_encode(x, px[0], px[1], pea[0], pea[1],
                        W_self, wmx, wme, W1[:H], W1[H:], b1)

    g1, g2 = _sc_gather_pairs(p1, p2, sm, dm)

    w2p = jnp.pad(W2, ((0, 0), (0, EDP - ED)))
    b2p = jnp.pad(b2, (0, EDP - ED))
    return _tc_loss(g1, g2, orig16, w2p, b2p)


# speedup vs baseline: 1.0771x; 1.0771x over previous
import functools

import jax
import jax.numpy as jnp
from jax import lax
from jax.experimental import pallas as pl
from jax.experimental.pallas import tpu as pltpu
from jax.experimental.pallas import tpu_sc as plsc

N = 10000
E = 320000
D = 128
H = 128
ED = 4
EDP = 8
M = 48000
NC = 2
NS = 16
NW = NC * NS
EROWS = E // 128
MROWS = M // 128
NP_ = 10112
RPT = NP_ // NS
ZR = 128
RCH = (128, 128, 128, 128, 120)
RCH_TAIL = (128, 128, 128, 128, 8)


def _sc_aggregate(x, src1d, dst1d, eaflat, corrflat, dm1d, z8):
    mesh = plsc.VectorSubcoreMesh(core_axis_name="c", subcore_axis_name="s")

    @functools.partial(
        pl.kernel,
        mesh=mesh,
        compiler_params=pltpu.CompilerParams(use_tc_tiling_on_sc=False,
                                             needs_layout_passes=False),
        out_type=[
            jax.ShapeDtypeStruct((NC, N, D), jnp.float32),
            jax.ShapeDtypeStruct((NC, N, EDP), jnp.float32),
        ],
        scratch_types=[
            pltpu.VMEM((128,), jnp.int32),
            pltpu.VMEM((128,), jnp.int32),
            pltpu.VMEM((512,), jnp.float32),
            pltpu.VMEM((128,), jnp.int32),
            pltpu.VMEM((128,), jnp.int32),
            pltpu.VMEM((512,), jnp.float32),
            pltpu.VMEM((128,), jnp.int32),
            pltpu.VMEM((128,), jnp.int32),
            pltpu.VMEM((512,), jnp.float32),
            pltpu.VMEM((128, EDP), jnp.float32),
            pltpu.VMEM((128, EDP), jnp.float32),
            pltpu.VMEM((128, EDP), jnp.float32),
            pltpu.VMEM((128, D), jnp.float32),
            pltpu.VMEM((128, D), jnp.float32),
            pltpu.VMEM_SHARED((NP_, D), jnp.float32),
            pltpu.VMEM_SHARED((NP_, EDP), jnp.float32),
            pltpu.SemaphoreType.DMA,
            pltpu.SemaphoreType.DMA,
            pltpu.SemaphoreType.DMA,
            pltpu.SemaphoreType.DMA,
            pltpu.SemaphoreType.DMA,
            pltpu.SemaphoreType.DMA,
            pltpu.SemaphoreType.DMA,
        ],
    )
    def k(x_hbm, src_hbm, dst_hbm, eaflat_hbm, corrflat_hbm, dm_hbm, z8_hbm,
          out_px, out_pea,
          srcv0, dstv0, east0, srcv1, dstv1, east1, srcv2, dstv2, east2,
          earows0, earows1, earows2, xrows0, xrows1,
          acc_x, acc_ea, semi0, semi1, semi2, semg0, semg1, sems0, sems1):
        c = lax.axis_index("c")
        s = lax.axis_index("s")
        wid = s * NC + c
        r0 = s * RPT

        idxb = ((srcv0, dstv0, east0, earows0, semi0),
                (srcv1, dstv1, east1, earows1, semi1),
                (srcv2, dstv2, east2, earows2, semi2))
        rowb = ((xrows0, semg0, sems0), (xrows1, semg1, sems1))

        iot = lax.iota(jnp.int32, 16)
        rowsub = lax.shift_right_logical(iot, 2)
        colsub = lax.bitwise_and(iot, 3)

        def expand(east, earows):
            for a in range(32):
                v = east[pl.ds(a * 16, 16)]
                plsc.store_scatter(earows, [a * 4 + rowsub, colsub], v)

        zvec = jnp.zeros((16,), jnp.float32)

        def zrow(r, carry):
            for cc in range(D // 16):
                xrows0[r, pl.ds(cc * 16, 16)] = zvec
            return carry

        lax.fori_loop(0, ZR, zrow, 0)
        pltpu.sync_copy(z8_hbm, earows0)
        pltpu.sync_copy(z8_hbm, earows1)
        pltpu.sync_copy(z8_hbm, earows2)

        off = 0
        for ch in RCH:
            pltpu.sync_copy(xrows0.at[pl.ds(0, ch)],
                            acc_x.at[pl.ds(r0 + off, ch)])
            pltpu.sync_copy(earows0.at[pl.ds(0, ch)],
                            acc_ea.at[pl.ds(r0 + off, ch)])
            off += ch
        plsc.subcore_barrier()

        def agg_issue_idx(t, j):
            rid = wid + NW * t

            @pl.when(rid < EROWS)
            def _():
                sv, dv, est, er, s_i = idxb[j]
                e0 = rid * 128
                pltpu.async_copy(src_hbm.at[pl.ds(e0, 128)], sv, s_i)
                pltpu.async_copy(dst_hbm.at[pl.ds(e0, 128)], dv, s_i)
                pltpu.async_copy(eaflat_hbm.at[pl.ds(e0 * 4, 512)], est, s_i)

        def agg_wait_idx(t, j):
            rid = wid + NW * t
            sv, dv, est, er, s_i = idxb[j]
            e0 = rid * 128
            pltpu.make_async_copy(src_hbm.at[pl.ds(e0, 128)], sv, s_i).wait()
            pltpu.make_async_copy(dst_hbm.at[pl.ds(e0, 128)], dv, s_i).wait()
            pltpu.make_async_copy(
                eaflat_hbm.at[pl.ds(e0 * 4, 512)], est, s_i).wait()

        agg_issue_idx(0, 0)
        agg_issue_idx(1, 1)
        agg_wait_idx(0, 0)
        pltpu.async_copy(x_hbm.at[srcv0], xrows0, semg0)

        def agg_group(g, carry):
            for u in range(6):
                t = 6 * g + u
                j, jn, j2, jp = u % 3, (u + 1) % 3, (u + 2) % 3, (u + 2) % 3
                b, nb = u % 2, (u + 1) % 2
                sv, dv, ea_, er, s_i = idxb[j]
                svn, dvn, ean, ern, s_in = idxb[jn]
                svp, dvp, eap, erp, s_ip = idxb[jp]
                xr, s_g, s_s = rowb[b]
                xrn, s_gn, s_sn = rowb[nb]
                rid = wid + NW * t

                @pl.when(rid < EROWS)
                def _():
                    expand(ea_, er)
                    pltpu.make_async_copy(x_hbm.at[sv], xr, s_g).wait()
                    pltpu.async_copy(xr, acc_x.at[dv], s_s, add=True)
                    pltpu.async_copy(er, acc_ea.at[dv], s_s, add=True)

                @pl.when((t >= 1) & (rid - NW < EROWS))
                def _():
                    pltpu.make_async_copy(xrn, acc_x.at[dvp], s_sn).wait()
                    pltpu.make_async_copy(erp, acc_ea.at[dvp], s_sn).wait()

                @pl.when(rid + NW < EROWS)
                def _():
                    agg_wait_idx(t + 1, jn)
                    pltpu.async_copy(x_hbm.at[svn], xrn, s_gn)

                agg_issue_idx(t + 2, j2)
            return carry

        lax.fori_loop(0, (EROWS // NW + 6) // 6 + 1, agg_group, 0)

        def m_issue_idx(t, j):
            rid = wid + NW * t

            @pl.when(rid < MROWS)
            def _():
                sv, dv, est, er, s_i = idxb[j]
                e0 = rid * 128
                pltpu.async_copy(dm_hbm.at[pl.ds(e0, 128)], dv, s_i)
                pltpu.async_copy(corrflat_hbm.at[pl.ds(e0 * 4, 512)], est, s_i)

        def m_wait_idx(t, j):
            rid = wid + NW * t
            sv, dv, est, er, s_i = idxb[j]
            e0 = rid * 128
            pltpu.make_async_copy(dm_hbm.at[pl.ds(e0, 128)], dv, s_i).wait()
            pltpu.make_async_copy(
                corrflat_hbm.at[pl.ds(e0 * 4, 512)], est, s_i).wait()

        m_issue_idx(0, 0)
        m_issue_idx(1, 1)

        def mask_group(g, carry):
            for u in range(6):
                t = 6 * g + u
                j, jn, j2, jp = u % 3, (u + 1) % 3, (u + 2) % 3, (u + 2) % 3
                b, nb = u % 2, (u + 1) % 2
                sv, dv, ea_, er, s_i = idxb[j]
                svp, dvp, eap, erp, s_ip = idxb[jp]
                xr, s_g, s_s = rowb[b]
                xrn, s_gn, s_sn = rowb[nb]
                rid = wid + NW * t

                @pl.when(rid < MROWS)
                def _():
                    m_wait_idx(t, j)
                    expand(ea_, er)
                    pltpu.async_copy(er, acc_ea.at[dv], s_s, add=True)

                @pl.when((t >= 1) & (rid - NW < MROWS))
                def _():
                    pltpu.make_async_copy(erp, acc_ea.at[dvp], s_sn).wait()

                m_issue_idx(t + 2, j2)
            return carry

        lax.fori_loop(0, (MROWS // NW + 6) // 6 + 1, mask_group, 0)

        plsc.subcore_barrier()

        @pl.when(s < NS - 1)
        def _():
            off = 0
            for ch in RCH:
                pltpu.sync_copy(acc_x.at[pl.ds(r0 + off, ch)],
                                xrows0.at[pl.ds(0, ch)])
                pltpu.sync_copy(xrows0.at[pl.ds(0, ch)],
                                out_px.at[c, pl.ds(r0 + off, ch)])
                pltpu.sync_copy(acc_ea.at[pl.ds(r0 + off, ch)],
                                earows0.at[pl.ds(0, ch)])
                pltpu.sync_copy(earows0.at[pl.ds(0, ch)],
                                out_pea.at[c, pl.ds(r0 + off, ch)])
                off += ch

        @pl.when(s == NS - 1)
        def _():
            off = 0
            for ch in RCH_TAIL:
                pltpu.sync_copy(acc_x.at[pl.ds(r0 + off, ch)],
                                xrows0.at[pl.ds(0, ch)])
                pltpu.sync_copy(xrows0.at[pl.ds(0, ch)],
                                out_px.at[c, pl.ds(r0 + off, ch)])
                pltpu.sync_copy(acc_ea.at[pl.ds(r0 + off, ch)],
                                earows0.at[pl.ds(0, ch)])
                pltpu.sync_copy(earows0.at[pl.ds(0, ch)],
                                out_pea.at[c, pl.ds(r0 + off, ch)])
                off += ch

    return k(x, src1d, dst1d, eaflat, corrflat, dm1d, z8)


def _sc_gather_pairs(p1, p2, sm1d, dm1d):
    mesh = plsc.VectorSubcoreMesh(core_axis_name="c", subcore_axis_name="s")

    @functools.partial(
        pl.kernel,
        mesh=mesh,
        out_type=[
            jax.ShapeDtypeStruct((M, H), jnp.float32),
            jax.ShapeDtypeStruct((M, H), jnp.float32),
        ],
        scratch_types=[
            pltpu.VMEM((128,), jnp.int32),
            pltpu.VMEM((128,), jnp.int32),
            pltpu.VMEM((128,), jnp.int32),
            pltpu.VMEM((128,), jnp.int32),
            pltpu.VMEM((128,), jnp.int32),
            pltpu.VMEM((128,), jnp.int32),
            pltpu.VMEM((128, H), jnp.float32),
            pltpu.VMEM((128, H), jnp.float32),
            pltpu.VMEM((128, H), jnp.float32),
            pltpu.VMEM((128, H), jnp.float32),
            pltpu.SemaphoreType.DMA,
            pltpu.SemaphoreType.DMA,
            pltpu.SemaphoreType.DMA,
            pltpu.SemaphoreType.DMA,
            pltpu.SemaphoreType.DMA,
            pltpu.SemaphoreType.DMA,
            pltpu.SemaphoreType.DMA,
        ],
    )
    def k(p1_hbm, p2_hbm, sm_hbm, dm_hbm, g1_out, g2_out,
          smv0, dmv0, smv1, dmv1, smv2, dmv2, r10, r20, r11, r21,
          semi0, semi1, semi2, semg0, semg1, semw0, semw1):
        c = lax.axis_index("c")
        s = lax.axis_index("s")
        wid = s * NC + c

        idxb = ((smv0, dmv0, semi0), (smv1, dmv1, semi1), (smv2, dmv2, semi2))
        rowb = ((r10, r20, semg0, semw0), (r11, r21, semg1, semw1))

        def issue_idx(t, j):
            rid = wid + NW * t

            @pl.when(rid < MROWS)
            def _():
                sm_, dm_, s_i = idxb[j]
                e0 = rid * 128
                pltpu.async_copy(sm_hbm.at[pl.ds(e0, 128)], sm_, s_i)
                pltpu.async_copy(dm_hbm.at[pl.ds(e0, 128)], dm_, s_i)

        def wait_idx(t, j):
            rid = wid + NW * t
            sm_, dm_, s_i = idxb[j]
            e0 = rid * 128
            pltpu.make_async_copy(sm_hbm.at[pl.ds(e0, 128)], sm_, s_i).wait()
            pltpu.make_async_copy(dm_hbm.at[pl.ds(e0, 128)], dm_, s_i).wait()

        def issue_gather(j, b):
            sm_, dm_, s_i = idxb[j]
            r1_, r2_, s_g, s_w = rowb[b]
            pltpu.async_copy(p1_hbm.at[sm_], r1_, s_g)
            pltpu.async_copy(p2_hbm.at[dm_], r2_, s_g)

        issue_idx(0, 0)
        issue_idx(1, 1)
        wait_idx(0, 0)
        issue_gather(0, 0)

        def group(g, carry):
            for u in range(6):
                t = 6 * g + u
                j, jn, j2 = u % 3, (u + 1) % 3, (u + 2) % 3
                b, nb = u % 2, (u + 1) % 2
                sm_, dm_, s_i = idxb[j]
                smn, dmn, s_in = idxb[jn]
                r1_, r2_, s_g, s_w = rowb[b]
                r1n, r2n, s_gn, s_wn = rowb[nb]
                rid = wid + NW * t
                e0 = rid * 128

                @pl.when(rid < MROWS)
                def _():
                    pltpu.make_async_copy(p1_hbm.at[sm_], r1_, s_g).wait()
                    pltpu.make_async_copy(p2_hbm.at[dm_], r2_, s_g).wait()
                    pltpu.async_copy(r1_, g1_out.at[pl.ds(e0, 128)], s_w)
                    pltpu.async_copy(r2_, g2_out.at[pl.ds(e0, 128)], s_w)

                @pl.when((t >= 1) & (rid - NW < MROWS))
                def _():
                    pe0 = (rid - NW) * 128
                    pltpu.make_async_copy(
                        r1n, g1_out.at[pl.ds(pe0, 128)], s_wn).wait()
                    pltpu.make_async_copy(
                        r2n, g2_out.at[pl.ds(pe0, 128)], s_wn).wait()

                @pl.when(rid + NW < MROWS)
                def _():
                    wait_idx(t + 1, jn)
                    issue_gather(jn, nb)

                issue_idx(t + 2, j2)
            return carry

        lax.fori_loop(0, (MROWS // NW + 6) // 6 + 1, group, 0)

    return k(p1, p2, sm1d, dm1d)


def _tc_encode(x, px, pea, w_self, wmx, wme, w1a, w1b, b1):
    BN = 1000

    def body(x_r, px_r, pea_r, ws_r, wmx_r, wme_r, w1a_r, w1b_r, b1_r,
             p1_o, p2_o):
        aggx = px_r[0] + px_r[1]
        aggea = pea_r[0] + pea_r[1]
        z = (jnp.dot(x_r[...], ws_r[...], preferred_element_type=jnp.float32)
             + jnp.dot(aggx, wmx_r[...], preferred_element_type=jnp.float32)
             + jnp.dot(aggea, wme_r[...], preferred_element_type=jnp.float32))
        emb = jnp.maximum(z, 0.0)
        p1_o[...] = (jnp.dot(emb, w1a_r[...], preferred_element_type=jnp.float32)
                     + b1_r[...])
        p2_o[...] = jnp.dot(emb, w1b_r[...], preferred_element_type=jnp.float32)

    row = lambda i: (i, 0)
    row3 = lambda i: (0, i, 0)
    fix = lambda i: (0, 0)
    return pl.pallas_call(
        body,
        grid=(N // BN,),
        in_specs=[
            pl.BlockSpec((BN, D), row),
            pl.BlockSpec((NC, BN, D), row3),
            pl.BlockSpec((NC, BN, EDP), row3),
            pl.BlockSpec((D, H), fix),
            pl.BlockSpec((D, H), fix),
            pl.BlockSpec((EDP, H), fix),
            pl.BlockSpec((H, H), fix),
            pl.BlockSpec((H, H), fix),
            pl.BlockSpec((1, H), fix),
        ],
        out_specs=[
            pl.BlockSpec((BN, H), row),
            pl.BlockSpec((BN, H), row),
        ],
        out_shape=[
            jax.ShapeDtypeStruct((N, H), jnp.float32),
            jax.ShapeDtypeStruct((N, H), jnp.float32),
        ],
    )(x, px, pea, w_self, wmx, wme, w1a, w1b, b1.reshape(1, H))


def _tc_loss(g1, g2, orig4, w2, b2):
    BM = 4800
    scale = 1.0 / (M * ED)

    def body(g1_r, g2_r, o_r, w2_r, b2_r, out_ref):
        i = pl.program_id(0)
        h = jnp.maximum(g1_r[...] + g2_r[...], 0.0)
        pred = (jnp.dot(h, w2_r[...], preferred_element_type=jnp.float32)
                + b2_r[...])
        part = jnp.sum(jnp.abs(pred - o_r[...])) * scale

        @pl.when(i == 0)
        def _():
            out_ref[0, 0] = part

        @pl.when(i > 0)
        def _():
            out_ref[0, 0] += part

    row = lambda i: (i, 0)
    fix = lambda i: (0, 0)
    out = pl.pallas_call(
        body,
        grid=(M // BM,),
        in_specs=[
            pl.BlockSpec((BM, H), row),
            pl.BlockSpec((BM, H), row),
            pl.BlockSpec((BM, ED), row),
            pl.BlockSpec((H, ED), fix),
            pl.BlockSpec((1, ED), fix),
        ],
        out_specs=pl.BlockSpec((1, 1), fix, memory_space=pltpu.SMEM),
        out_shape=jax.ShapeDtypeStruct((1, 1), jnp.float32),
    )(g1, g2, orig4, w2, b2.reshape(1, ED))
    return out[0, 0]


def kernel(x, edge_index, edge_attr, mask_indices, edge_mask_token,
           W_self, W_msg, W1, b1, W2, b2):
    src = edge_index[0]
    dst = edge_index[1]
    eaflat = lax.optimization_barrier(edge_attr.reshape(E * ED))
    origflat = jnp.take(
        eaflat,
        jnp.repeat(mask_indices * ED, ED)
        + jnp.tile(jnp.arange(ED, dtype=jnp.int32), M),
        mode="clip")
    corrflat = jnp.tile(edge_mask_token, E) - eaflat
    z8 = jnp.zeros((128, EDP), jnp.float32)
    sm = jnp.take(src, mask_indices, mode="clip")
    dm = jnp.take(dst, mask_indices, mode="clip")

    px, pea = _sc_aggregate(x, src, dst, eaflat, corrflat, dm, z8)

    wmx = W_msg[:D]
    wme = jnp.pad(W_msg[D:], ((0, EDP - ED), (0, 0)))
    p1, p2 = _tc_encode(x, px, pea, W_self, wmx, wme, W1[:H], W1[H:], b1)

    g1, g2 = _sc_gather_pairs(p1, p2, sm, dm)

    return _tc_loss(g1, g2, origflat.reshape(M, ED), W2, b2)


# speedup vs baseline: 1.2956x; 1.2029x over previous
import functools

import jax
import jax.numpy as jnp
from jax import lax
from jax.experimental import pallas as pl
from jax.experimental.pallas import tpu as pltpu
from jax.experimental.pallas import tpu_sc as plsc

N = 10000
E = 320000
D = 128
H = 128
ED = 4
EDP = 8
M = 48000
NC = 2
NS = 16
NW = NC * NS
EROWS = E // 128
MROWS = M // 128
NP_ = 10112
RPT = NP_ // NS
ZR = 128
RCH = (128, 128, 128, 128, 120)
RCH_TAIL = (128, 128, 128, 128, 8)


def _sc_aggregate(x, src1d, dst1d, eaflat, corrflat, dm1d, z8):
    mesh = plsc.VectorSubcoreMesh(core_axis_name="c", subcore_axis_name="s")

    @functools.partial(
        pl.kernel,
        mesh=mesh,
        compiler_params=pltpu.CompilerParams(use_tc_tiling_on_sc=False,
                                             needs_layout_passes=False),
        out_type=[
            jax.ShapeDtypeStruct((NC, N, D), jnp.float32),
            jax.ShapeDtypeStruct((NC, N, EDP), jnp.float32),
        ],
        scratch_types=[
            pltpu.VMEM((128,), jnp.int32),
            pltpu.VMEM((128,), jnp.int32),
            pltpu.VMEM((512,), jnp.float32),
            pltpu.VMEM((128,), jnp.int32),
            pltpu.VMEM((128,), jnp.int32),
            pltpu.VMEM((512,), jnp.float32),
            pltpu.VMEM((128,), jnp.int32),
            pltpu.VMEM((128,), jnp.int32),
            pltpu.VMEM((512,), jnp.float32),
            pltpu.VMEM((128, EDP), jnp.float32),
            pltpu.VMEM((128, EDP), jnp.float32),
            pltpu.VMEM((128, EDP), jnp.float32),
            pltpu.VMEM((128, D), jnp.float32),
            pltpu.VMEM((128, D), jnp.float32),
            pltpu.VMEM_SHARED((NP_, D), jnp.float32),
            pltpu.VMEM_SHARED((NP_, EDP), jnp.float32),
            pltpu.SemaphoreType.DMA,
            pltpu.SemaphoreType.DMA,
            pltpu.SemaphoreType.DMA,
            pltpu.SemaphoreType.DMA,
            pltpu.SemaphoreType.DMA,
            pltpu.SemaphoreType.DMA,
            pltpu.SemaphoreType.DMA,
        ],
    )
    def k(x_hbm, src_hbm, dst_hbm, eaflat_hbm, corrflat_hbm, dm_hbm, z8_hbm,
          out_px, out_pea,
          srcv0, dstv0, east0, srcv1, dstv1, east1, srcv2, dstv2, east2,
          earows0, earows1, earows2, xrows0, xrows1,
          acc_x, acc_ea, semi0, semi1, semi2, semg0, semg1, sems0, sems1):
        c = lax.axis_index("c")
        s = lax.axis_index("s")
        wid = s * NC + c
        r0 = s * RPT

        idxb = ((srcv0, dstv0, east0, earows0, semi0),
                (srcv1, dstv1, east1, earows1, semi1),
                (srcv2, dstv2, east2, earows2, semi2))
        rowb = ((xrows0, semg0, sems0), (xrows1, semg1, sems1))

        iot = lax.iota(jnp.int32, 16)
        rowsub = lax.shift_right_logical(iot, 2)
        colsub = lax.bitwise_and(iot, 3)

        def expand(east, earows):
            for a in range(32):
                v = east[pl.ds(a * 16, 16)]
                plsc.store_scatter(earows, [a * 4 + rowsub, colsub], v)

        zvec = jnp.zeros((16,), jnp.float32)

        def zrow(r, carry):
            for cc in range(D // 16):
                xrows0[r, pl.ds(cc * 16, 16)] = zvec
            return carry

        lax.fori_loop(0, ZR, zrow, 0)
        pltpu.sync_copy(z8_hbm, earows0)
        pltpu.sync_copy(z8_hbm, earows1)
        pltpu.sync_copy(z8_hbm, earows2)

        off = 0
        for ch in RCH:
            pltpu.sync_copy(xrows0.at[pl.ds(0, ch)],
                            acc_x.at[pl.ds(r0 + off, ch)])
            pltpu.sync_copy(earows0.at[pl.ds(0, ch)],
                            acc_ea.at[pl.ds(r0 + off, ch)])
            off += ch
        plsc.subcore_barrier()

        def agg_issue_idx(t, j):
            rid = wid + NW * t

            @pl.when(rid < EROWS)
            def _():
                sv, dv, est, er, s_i = idxb[j]
                e0 = rid * 128
                pltpu.async_copy(src_hbm.at[pl.ds(e0, 128)], sv, s_i)
                pltpu.async_copy(dst_hbm.at[pl.ds(e0, 128)], dv, s_i)
                pltpu.async_copy(eaflat_hbm.at[pl.ds(e0 * 4, 512)], est, s_i)

        def agg_wait_idx(t, j):
            rid = wid + NW * t
            sv, dv, est, er, s_i = idxb[j]
            e0 = rid * 128
            pltpu.make_async_copy(src_hbm.at[pl.ds(e0, 128)], sv, s_i).wait()
            pltpu.make_async_copy(dst_hbm.at[pl.ds(e0, 128)], dv, s_i).wait()
            pltpu.make_async_copy(
                eaflat_hbm.at[pl.ds(e0 * 4, 512)], est, s_i).wait()

        agg_issue_idx(0, 0)
        agg_issue_idx(1, 1)
        agg_wait_idx(0, 0)
        pltpu.async_copy(x_hbm.at[srcv0], xrows0, semg0)

        def agg_group(g, carry):
            for u in range(6):
                t = 6 * g + u
                j, jn, j2, jp = u % 3, (u + 1) % 3, (u + 2) % 3, (u + 2) % 3
                b, nb = u % 2, (u + 1) % 2
                sv, dv, ea_, er, s_i = idxb[j]
                svn, dvn, ean, ern, s_in = idxb[jn]
                svp, dvp, eap, erp, s_ip = idxb[jp]
                xr, s_g, s_s = rowb[b]
                xrn, s_gn, s_sn = rowb[nb]
                rid = wid + NW * t

                @pl.when(rid < EROWS)
                def _():
                    expand(ea_, er)
                    pltpu.make_async_copy(x_hbm.at[sv], xr, s_g).wait()
                    pltpu.async_copy(xr, acc_x.at[dv], s_s, add=True)
                    pltpu.async_copy(er, acc_ea.at[dv], s_s, add=True)

                @pl.when((t >= 1) & (rid - NW < EROWS))
                def _():
                    pltpu.make_async_copy(xrn, acc_x.at[dvp], s_sn).wait()
                    pltpu.make_async_copy(erp, acc_ea.at[dvp], s_sn).wait()

                @pl.when(rid + NW < EROWS)
                def _():
                    agg_wait_idx(t + 1, jn)
                    pltpu.async_copy(x_hbm.at[svn], xrn, s_gn)

                agg_issue_idx(t + 2, j2)
            return carry

        lax.fori_loop(0, (EROWS // NW + 6) // 6 + 1, agg_group, 0)

        def m_issue_idx(t, j):
            rid = wid + NW * t

            @pl.when(rid < MROWS)
            def _():
                sv, dv, est, er, s_i = idxb[j]
                e0 = rid * 128
                pltpu.async_copy(dm_hbm.at[pl.ds(e0, 128)], dv, s_i)
                pltpu.async_copy(corrflat_hbm.at[pl.ds(e0 * 4, 512)], est, s_i)

        def m_wait_idx(t, j):
            rid = wid + NW * t
            sv, dv, est, er, s_i = idxb[j]
            e0 = rid * 128
            pltpu.make_async_copy(dm_hbm.at[pl.ds(e0, 128)], dv, s_i).wait()
            pltpu.make_async_copy(
                corrflat_hbm.at[pl.ds(e0 * 4, 512)], est, s_i).wait()

        m_issue_idx(0, 0)
        m_issue_idx(1, 1)

        def mask_group(g, carry):
            for u in range(6):
                t = 6 * g + u
                j, jn, j2, jp = u % 3, (u + 1) % 3, (u + 2) % 3, (u + 2) % 3
                b, nb = u % 2, (u + 1) % 2
                sv, dv, ea_, er, s_i = idxb[j]
                svp, dvp, eap, erp, s_ip = idxb[jp]
                xr, s_g, s_s = rowb[b]
                xrn, s_gn, s_sn = rowb[nb]
                rid = wid + NW * t

                @pl.when(rid < MROWS)
                def _():
                    m_wait_idx(t, j)
                    expand(ea_, er)
                    pltpu.async_copy(er, acc_ea.at[dv], s_s, add=True)

                @pl.when((t >= 1) & (rid - NW < MROWS))
                def _():
                    pltpu.make_async_copy(erp, acc_ea.at[dvp], s_sn).wait()

                m_issue_idx(t + 2, j2)
            return carry

        lax.fori_loop(0, (MROWS // NW + 6) // 6 + 1, mask_group, 0)

        plsc.subcore_barrier()

        @pl.when(s < NS - 1)
        def _():
            off = 0
            for ch in RCH:
                pltpu.sync_copy(acc_x.at[pl.ds(r0 + off, ch)],
                                xrows0.at[pl.ds(0, ch)])
                pltpu.sync_copy(xrows0.at[pl.ds(0, ch)],
                                out_px.at[c, pl.ds(r0 + off, ch)])
                pltpu.sync_copy(acc_ea.at[pl.ds(r0 + off, ch)],
                                earows0.at[pl.ds(0, ch)])
                pltpu.sync_copy(earows0.at[pl.ds(0, ch)],
                                out_pea.at[c, pl.ds(r0 + off, ch)])
                off += ch

        @pl.when(s == NS - 1)
        def _():
            off = 0
            for ch in RCH_TAIL:
                pltpu.sync_copy(acc_x.at[pl.ds(r0 + off, ch)],
                                xrows0.at[pl.ds(0, ch)])
                pltpu.sync_copy(xrows0.at[pl.ds(0, ch)],
                                out_px.at[c, pl.ds(r0 + off, ch)])
                pltpu.sync_copy(acc_ea.at[pl.ds(r0 + off, ch)],
                                earows0.at[pl.ds(0, ch)])
                pltpu.sync_copy(earows0.at[pl.ds(0, ch)],
                                out_pea.at[c, pl.ds(r0 + off, ch)])
                off += ch

    return k(x, src1d, dst1d, eaflat, corrflat, dm1d, z8)


def _sc_gather_pairs(p1, p2, sm1d, dm1d):
    mesh = plsc.VectorSubcoreMesh(core_axis_name="c", subcore_axis_name="s")

    @functools.partial(
        pl.kernel,
        mesh=mesh,
        out_type=[
            jax.ShapeDtypeStruct((M, H), jnp.float32),
            jax.ShapeDtypeStruct((M, H), jnp.float32),
        ],
        scratch_types=[
            pltpu.VMEM((128,), jnp.int32),
            pltpu.VMEM((128,), jnp.int32),
            pltpu.VMEM((128,), jnp.int32),
            pltpu.VMEM((128,), jnp.int32),
            pltpu.VMEM((128,), jnp.int32),
            pltpu.VMEM((128,), jnp.int32),
            pltpu.VMEM((128, H), jnp.float32),
            pltpu.VMEM((128, H), jnp.float32),
            pltpu.VMEM((128, H), jnp.float32),
            pltpu.VMEM((128, H), jnp.float32),
            pltpu.SemaphoreType.DMA,
            pltpu.SemaphoreType.DMA,
            pltpu.SemaphoreType.DMA,
            pltpu.SemaphoreType.DMA,
            pltpu.SemaphoreType.DMA,
            pltpu.SemaphoreType.DMA,
            pltpu.SemaphoreType.DMA,
        ],
    )
    def k(p1_hbm, p2_hbm, sm_hbm, dm_hbm, g1_out, g2_out,
          smv0, dmv0, smv1, dmv1, smv2, dmv2, r10, r20, r11, r21,
          semi0, semi1, semi2, semg0, semg1, semw0, semw1):
        c = lax.axis_index("c")
        s = lax.axis_index("s")
        wid = s * NC + c

        idxb = ((smv0, dmv0, semi0), (smv1, dmv1, semi1), (smv2, dmv2, semi2))
        rowb = ((r10, r20, semg0, semw0), (r11, r21, semg1, semw1))

        def issue_idx(t, j):
            rid = wid + NW * t

            @pl.when(rid < MROWS)
            def _():
                sm_, dm_, s_i = idxb[j]
                e0 = rid * 128
                pltpu.async_copy(sm_hbm.at[pl.ds(e0, 128)], sm_, s_i)
                pltpu.async_copy(dm_hbm.at[pl.ds(e0, 128)], dm_, s_i)

        def wait_idx(t, j):
            rid = wid + NW * t
            sm_, dm_, s_i = idxb[j]
            e0 = rid * 128
            pltpu.make_async_copy(sm_hbm.at[pl.ds(e0, 128)], sm_, s_i).wait()
            pltpu.make_async_copy(dm_hbm.at[pl.ds(e0, 128)], dm_, s_i).wait()

        def issue_gather(j, b):
            sm_, dm_, s_i = idxb[j]
            r1_, r2_, s_g, s_w = rowb[b]
            pltpu.async_copy(p1_hbm.at[sm_], r1_, s_g)
            pltpu.async_copy(p2_hbm.at[dm_], r2_, s_g)

        issue_idx(0, 0)
        issue_idx(1, 1)
        wait_idx(0, 0)
        issue_gather(0, 0)

        def group(g, carry):
            for u in range(6):
                t = 6 * g + u
                j, jn, j2 = u % 3, (u + 1) % 3, (u + 2) % 3
                b, nb = u % 2, (u + 1) % 2
                sm_, dm_, s_i = idxb[j]
                smn, dmn, s_in = idxb[jn]
                r1_, r2_, s_g, s_w = rowb[b]
                r1n, r2n, s_gn, s_wn = rowb[nb]
                rid = wid + NW * t
                e0 = rid * 128

                @pl.when(rid < MROWS)
                def _():
                    pltpu.make_async_copy(p1_hbm.at[sm_], r1_, s_g).wait()
                    pltpu.make_async_copy(p2_hbm.at[dm_], r2_, s_g).wait()
                    pltpu.async_copy(r1_, g1_out.at[pl.ds(e0, 128)], s_w)
                    pltpu.async_copy(r2_, g2_out.at[pl.ds(e0, 128)], s_w)

                @pl.when((t >= 1) & (rid - NW < MROWS))
                def _():
                    pe0 = (rid - NW) * 128
                    pltpu.make_async_copy(
                        r1n, g1_out.at[pl.ds(pe0, 128)], s_wn).wait()
                    pltpu.make_async_copy(
                        r2n, g2_out.at[pl.ds(pe0, 128)], s_wn).wait()

                @pl.when(rid + NW < MROWS)
                def _():
                    wait_idx(t + 1, jn)
                    issue_gather(jn, nb)

                issue_idx(t + 2, j2)
            return carry

        lax.fori_loop(0, (MROWS // NW + 6) // 6 + 1, group, 0)

    return k(p1, p2, sm1d, dm1d)


def _tc_encode(x, px, pea, w_self, wmx, wme, w1a, w1b, b1):
    BN = 1000

    def body(x_r, px_r, pea_r, ws_r, wmx_r, wme_r, w1a_r, w1b_r, b1_r,
             p1_o, p2_o):
        aggx = px_r[0] + px_r[1]
        aggea = pea_r[0] + pea_r[1]
        z = (jnp.dot(x_r[...], ws_r[...], preferred_element_type=jnp.float32)
             + jnp.dot(aggx, wmx_r[...], preferred_element_type=jnp.float32)
             + jnp.dot(aggea, wme_r[...], preferred_element_type=jnp.float32))
        emb = jnp.maximum(z, 0.0)
        p1_o[...] = (jnp.dot(emb, w1a_r[...], preferred_element_type=jnp.float32)
                     + b1_r[...])
        p2_o[...] = jnp.dot(emb, w1b_r[...], preferred_element_type=jnp.float32)

    row = lambda i: (i, 0)
    row3 = lambda i: (0, i, 0)
    fix = lambda i: (0, 0)
    return pl.pallas_call(
        body,
        grid=(N // BN,),
        in_specs=[
            pl.BlockSpec((BN, D), row),
            pl.BlockSpec((NC, BN, D), row3),
            pl.BlockSpec((NC, BN, EDP), row3),
            pl.BlockSpec((D, H), fix),
            pl.BlockSpec((D, H), fix),
            pl.BlockSpec((EDP, H), fix),
            pl.BlockSpec((H, H), fix),
            pl.BlockSpec((H, H), fix),
            pl.BlockSpec((1, H), fix),
        ],
        out_specs=[
            pl.BlockSpec((BN, H), row),
            pl.BlockSpec((BN, H), row),
        ],
        out_shape=[
            jax.ShapeDtypeStruct((N, H), jnp.float32),
            jax.ShapeDtypeStruct((N, H), jnp.float32),
        ],
    )(x, px, pea, w_self, wmx, wme, w1a, w1b, b1.reshape(1, H))


def _tc_loss(g1, g2, orig4, w2, b2):
    BM = 4800
    scale = 1.0 / (M * ED)

    def body(g1_r, g2_r, o_r, w2_r, b2_r, out_ref):
        i = pl.program_id(0)
        h = jnp.maximum(g1_r[...] + g2_r[...], 0.0)
        pred = (jnp.dot(h, w2_r[...], preferred_element_type=jnp.float32)
                + b2_r[...])
        part = jnp.sum(jnp.abs(pred - o_r[...])) * scale

        @pl.when(i == 0)
        def _():
            out_ref[0, 0] = part

        @pl.when(i > 0)
        def _():
            out_ref[0, 0] += part

    row = lambda i: (i, 0)
    fix = lambda i: (0, 0)
    out = pl.pallas_call(
        body,
        grid=(M // BM,),
        in_specs=[
            pl.BlockSpec((BM, H), row),
            pl.BlockSpec((BM, H), row),
            pl.BlockSpec((BM, ED), row),
            pl.BlockSpec((H, ED), fix),
            pl.BlockSpec((1, ED), fix),
        ],
        out_specs=pl.BlockSpec((1, 1), fix, memory_space=pltpu.SMEM),
        out_shape=jax.ShapeDtypeStruct((1, 1), jnp.float32),
    )(g1, g2, orig4, w2, b2.reshape(1, ED))
    return out[0, 0]


def kernel(x, edge_index, edge_attr, mask_indices, edge_mask_token,
           W_self, W_msg, W1, b1, W2, b2):
    src = edge_index[0]
    dst = edge_index[1]
    eaflat = lax.optimization_barrier(edge_attr.reshape(E * ED))
    origflat = jnp.take(
        eaflat,
        jnp.repeat(mask_indices * ED, ED)
        + jnp.tile(jnp.arange(ED, dtype=jnp.int32), M),
        mode="clip")
    corrflat = jnp.tile(edge_mask_token, M) - origflat
    z8 = jnp.zeros((128, EDP), jnp.float32)
    sm = jnp.take(src, mask_indices, mode="clip")
    dm = jnp.take(dst, mask_indices, mode="clip")

    px, pea = _sc_aggregate(x, src, dst, eaflat, corrflat, dm, z8)

    wmx = W_msg[:D]
    wme = jnp.pad(W_msg[D:], ((0, EDP - ED), (0, 0)))
    p1, p2 = _tc_encode(x, px, pea, W_self, wmx, wme, W1[:H], W1[H:], b1)

    g1, g2 = _sc_gather_pairs(p1, p2, sm, dm)

    return _tc_loss(g1, g2, origflat.reshape(M, ED), W2, b2)
